# baseline (speedup 1.0000x reference)
import functools

import numpy as np

import jax
import jax.numpy as jnp
from jax import lax
from jax.experimental import pallas as pl
from jax.experimental.pallas import tpu as pltpu


def _col_masks(H, W):
    HW = H * W
    col = np.arange(HW) % W
    m = np.ones((2, HW), np.float32)
    m[0, col == W - 1] = 0.0
    m[1, col == 0] = 0.0
    return m


def _row_masks(H, W):
    HW = H * W
    m = np.ones((2, HW), np.float32)
    m[0, :W] = 0.0
    m[1, HW - W:] = 0.0
    return m


def _conv_stats_kernel(x_ref, w_ref, cm_ref, rm_ref, o_ref, s_ref, *, H, W):
    HW = H * W
    Cin = x_ref.shape[1]
    x = x_ref[0].reshape(Cin, HW).astype(jnp.bfloat16)
    xl = x * cm_ref[0:1, :]
    xr = x * cm_ref[1:2, :]

    def group(dh):
        part = None
        for dw in (-1, 0, 1):
            src = (xl, x, xr)[dw + 1]
            shift = (-(dh * W + dw)) % HW
            tap = src if shift == 0 else pltpu.roll(src, shift, axis=1)
            t = (dh + 1) * 3 + (dw + 1)
            d = jnp.dot(w_ref[t], tap, preferred_element_type=jnp.float32)
            part = d if part is None else part + d
        return part

    acc = group(-1) * rm_ref[0:1, :]
    acc = acc + group(0)
    acc = acc + group(1) * rm_ref[1:2, :]

    o_ref[...] = acc.astype(jnp.bfloat16)[None]
    s_ref[0, :, 0:1] = jnp.sum(acc, axis=1, keepdims=True)
    s_ref[0, :, 1:2] = jnp.sum(acc * acc, axis=1, keepdims=True)


def _norm_relu_kernel(s_ref, beta_ref, y_ref, o_ref, *, inv_count, eps):
    H, W = o_ref.shape[2], o_ref.shape[3]
    tot = jnp.sum(s_ref[...], axis=0)
    mean = tot[:, 0:1] * inv_count
    var = jnp.maximum(tot[:, 1:2] * inv_count - mean * mean, 0.0)
    scale = lax.rsqrt(var + eps)
    shift = beta_ref[...] - mean * scale
    y = y_ref[0].astype(jnp.float32)
    out = jnp.maximum(y * scale + shift, 0.0)
    o_ref[...] = out.reshape(1, out.shape[0], H, W)


@jax.jit
def _conv_block(x_nchw, weight_oihw, beta):
    eps = 1e-5
    N, Cin, H, W = x_nchw.shape
    Cout = weight_oihw.shape[0]
    HW = H * W

    w_taps = jnp.transpose(weight_oihw.astype(jnp.float32),
                           (2, 3, 0, 1)).reshape(9, Cout, Cin)
    w_taps = w_taps.astype(jnp.bfloat16)
    cmasks = jnp.asarray(_col_masks(H, W), dtype=jnp.bfloat16)
    rmasks = jnp.asarray(_row_masks(H, W), dtype=jnp.float32)

    conv_flops = 2 * Cout * 9 * Cin * HW

    y_raw, stats = pl.pallas_call(
        functools.partial(_conv_stats_kernel, H=H, W=W),
        out_shape=(
            jax.ShapeDtypeStruct((N, Cout, HW), jnp.bfloat16),
            jax.ShapeDtypeStruct((N, Cout, 2), jnp.float32),
        ),
        grid=(N,),
        in_specs=[
            pl.BlockSpec((1, Cin, H, W), lambda n: (n, 0, 0, 0)),
            pl.BlockSpec((9, Cout, Cin), lambda n: (0, 0, 0)),
            pl.BlockSpec((2, HW), lambda n: (0, 0)),
            pl.BlockSpec((2, HW), lambda n: (0, 0)),
        ],
        out_specs=(
            pl.BlockSpec((1, Cout, HW), lambda n: (n, 0, 0)),
            pl.BlockSpec((1, Cout, 2), lambda n: (n, 0, 0)),
        ),
        compiler_params=pltpu.CompilerParams(
            dimension_semantics=("parallel",)),
        cost_estimate=pl.CostEstimate(
            flops=N * conv_flops,
            transcendentals=0,
            bytes_accessed=4 * N * Cin * HW + 2 * (9 * Cout * Cin + 2 * HW
                                + N * Cout * HW) + 4 * N * Cout * 2),
    )(x_nchw, w_taps, cmasks, rmasks)

    y = pl.pallas_call(
        functools.partial(_norm_relu_kernel,
                          inv_count=1.0 / float(N * HW), eps=eps),
        out_shape=jax.ShapeDtypeStruct((N, Cout, H, W), jnp.float32),
        grid=(N,),
        in_specs=[
            pl.BlockSpec((N, Cout, 2), lambda n: (0, 0, 0)),
            pl.BlockSpec((Cout, 1), lambda n: (0, 0)),
            pl.BlockSpec((1, Cout, HW), lambda n: (n, 0, 0)),
        ],
        out_specs=pl.BlockSpec((1, Cout, H, W), lambda n: (n, 0, 0, 0)),
        compiler_params=pltpu.CompilerParams(
            dimension_semantics=("parallel",)),
        cost_estimate=pl.CostEstimate(
            flops=2 * N * Cout * HW,
            transcendentals=Cout,
            bytes_accessed=2 * N * Cout * HW + 4 * N * Cout * HW
                           + 4 * (N * Cout * 2 + Cout)),
    )(stats, beta.astype(jnp.float32).reshape(Cout, 1), y_raw)

    return y


def kernel(x_nchw, weight_oihw, beta):
    return _conv_block(x_nchw, weight_oihw, beta)


# speedup vs baseline: 2.9031x; 2.9031x over previous
import functools

import numpy as np

import jax
import jax.numpy as jnp
from jax import lax
from jax.experimental import pallas as pl
from jax.experimental.pallas import tpu as pltpu


def _edge_masks(H, W, C):
    col = np.arange(H * W) % W
    m = np.ones((2, H * W, 1), np.float32)
    m[0, col == 0, 0] = 0.0
    m[1, col == W - 1, 0] = 0.0
    return np.broadcast_to(m, (2, H * W, C)).copy()


def _conv_stats_kernel(x_ref, w_ref, m_ref, o_ref, s_ref, *, H, W):
    HW = H * W
    Cin = x_ref.shape[3]
    Cout = w_ref.shape[2] // 3
    x2 = x_ref[0].reshape(HW, Cin)
    zrow = jnp.zeros((W, Cin), jnp.float32)
    s_up = jnp.concatenate([zrow, x2[0:HW - W]], axis=0).astype(jnp.bfloat16)
    s_mid = x2.astype(jnp.bfloat16)
    s_dn = jnp.concatenate([x2[W:HW], zrow], axis=0).astype(jnp.bfloat16)

    z = jnp.dot(s_up, w_ref[0], preferred_element_type=jnp.float32)
    z = z + jnp.dot(s_mid, w_ref[1], preferred_element_type=jnp.float32)
    z = z + jnp.dot(s_dn, w_ref[2], preferred_element_type=jnp.float32)

    acc = z[:, Cout:2 * Cout]
    acc = acc + pltpu.roll(z[:, 0:Cout], 1, axis=0) * m_ref[0]
    acc = acc + pltpu.roll(z[:, 2 * Cout:], HW - 1, axis=0) * m_ref[1]

    o_ref[...] = acc.astype(jnp.bfloat16)[None]
    s_ref[0, 0:1, :] = jnp.sum(acc, axis=0, keepdims=True)
    s_ref[0, 1:2, :] = jnp.sum(acc * acc, axis=0, keepdims=True)


def _norm_relu_kernel(s_ref, beta_ref, y_ref, o_ref, *, inv_count, eps):
    tot = jnp.sum(s_ref[...], axis=0)
    mean = tot[0:1, :] * inv_count
    var = jnp.maximum(tot[1:2, :] * inv_count - mean * mean, 0.0)
    scale = lax.rsqrt(var + eps)
    shift = beta_ref[...] - mean * scale
    y = y_ref[0].astype(jnp.float32)
    o_ref[...] = jnp.maximum(y * scale + shift, 0.0)[None]


@jax.jit
def _conv_block(x_nchw, weight_oihw, beta):
    eps = 1e-5
    N, Cin, H, W = x_nchw.shape
    Cout = weight_oihw.shape[0]
    HW = H * W

    x = jnp.transpose(x_nchw, (0, 2, 3, 1))
    w_cat = jnp.transpose(weight_oihw.astype(jnp.float32),
                          (2, 1, 3, 0)).reshape(3, Cin, 3 * Cout)
    w_cat = w_cat.astype(jnp.bfloat16)
    masks = jnp.asarray(_edge_masks(H, W, Cout), dtype=jnp.float32)

    conv_flops = 2 * Cout * 9 * Cin * HW

    y_raw, stats = pl.pallas_call(
        functools.partial(_conv_stats_kernel, H=H, W=W),
        out_shape=(
            jax.ShapeDtypeStruct((N, HW, Cout), jnp.bfloat16),
            jax.ShapeDtypeStruct((N, 2, Cout), jnp.float32),
        ),
        grid=(N,),
        in_specs=[
            pl.BlockSpec((1, H, W, Cin), lambda n: (n, 0, 0, 0)),
            pl.BlockSpec((3, Cin, 3 * Cout), lambda n: (0, 0, 0)),
            pl.BlockSpec((2, HW, Cout), lambda n: (0, 0, 0)),
        ],
        out_specs=(
            pl.BlockSpec((1, HW, Cout), lambda n: (n, 0, 0)),
            pl.BlockSpec((1, 2, Cout), lambda n: (n, 0, 0)),
        ),
        compiler_params=pltpu.CompilerParams(
            dimension_semantics=("parallel",)),
        cost_estimate=pl.CostEstimate(
            flops=N * conv_flops,
            transcendentals=0,
            bytes_accessed=4 * N * Cin * HW + 2 * (3 * Cin * 3 * Cout
                                + N * Cout * HW) + 4 * (2 * HW * Cout
                                + N * Cout * 2)),
    )(x, w_cat, masks)

    y = pl.pallas_call(
        functools.partial(_norm_relu_kernel,
                          inv_count=1.0 / float(N * HW), eps=eps),
        out_shape=jax.ShapeDtypeStruct((N, HW, Cout), jnp.float32),
        grid=(N,),
        in_specs=[
            pl.BlockSpec((N, 2, Cout), lambda n: (0, 0, 0)),
            pl.BlockSpec((1, Cout), lambda n: (0, 0)),
            pl.BlockSpec((1, HW, Cout), lambda n: (n, 0, 0)),
        ],
        out_specs=pl.BlockSpec((1, HW, Cout), lambda n: (n, 0, 0)),
        compiler_params=pltpu.CompilerParams(
            dimension_semantics=("parallel",)),
        cost_estimate=pl.CostEstimate(
            flops=2 * N * Cout * HW,
            transcendentals=Cout,
            bytes_accessed=2 * N * Cout * HW + 4 * N * Cout * HW
                           + 4 * (N * Cout * 2 + Cout)),
    )(stats, beta.astype(jnp.float32).reshape(1, Cout), y_raw)

    return jnp.transpose(y.reshape(N, H, W, Cout), (0, 3, 1, 2))


def kernel(x_nchw, weight_oihw, beta):
    return _conv_block(x_nchw, weight_oihw, beta)


# speedup vs baseline: 3.4286x; 1.1810x over previous
import functools

import numpy as np

import jax
import jax.numpy as jnp
from jax import lax
from jax.experimental import pallas as pl
from jax.experimental.pallas import tpu as pltpu


def _edge_masks(H, W, C):
    col = np.arange(H * W) % W
    m = np.ones((2, H * W, 1), np.float32)
    m[0, col == 0, 0] = 0.0
    m[1, col == W - 1, 0] = 0.0
    return np.broadcast_to(m, (2, H * W, C)).copy()


def _conv_stats_kernel(x_ref, w_ref, m_ref, o_ref, s_ref, x3_ref, *, H, W):
    HW = H * W
    Cin = x_ref.shape[3]
    Cout = w_ref.shape[1] // 3
    x2 = x_ref[0].reshape(HW, Cin)
    zrow = jnp.zeros((W, Cin), jnp.float32)
    x3_ref[:, 0:Cin] = jnp.concatenate(
        [zrow, x2[0:HW - W]], axis=0).astype(jnp.bfloat16)
    x3_ref[:, Cin:2 * Cin] = x2.astype(jnp.bfloat16)
    x3_ref[:, 2 * Cin:] = jnp.concatenate(
        [x2[W:HW], zrow], axis=0).astype(jnp.bfloat16)

    z = jnp.dot(x3_ref[...], w_ref[...], preferred_element_type=jnp.float32)

    acc = z[:, Cout:2 * Cout]
    acc = acc + pltpu.roll(z[:, 0:Cout], 1, axis=0) * m_ref[0]
    acc = acc + pltpu.roll(z[:, 2 * Cout:], HW - 1, axis=0) * m_ref[1]

    o_ref[...] = acc.astype(jnp.bfloat16)[None]
    s_ref[0, 0:1, :] = jnp.sum(acc, axis=0, keepdims=True)
    s_ref[0, 1:2, :] = jnp.sum(acc * acc, axis=0, keepdims=True)


def _norm_relu_kernel(s_ref, beta_ref, y_ref, o_ref, *, inv_count, eps):
    tot = jnp.sum(s_ref[...], axis=0)
    mean = tot[0:1, :] * inv_count
    var = jnp.maximum(tot[1:2, :] * inv_count - mean * mean, 0.0)
    scale = lax.rsqrt(var + eps)
    shift = beta_ref[...] - mean * scale
    y = y_ref[0].astype(jnp.float32)
    o_ref[...] = jnp.maximum(y * scale + shift, 0.0)[None]


@jax.jit
def _conv_block(x_nchw, weight_oihw, beta):
    eps = 1e-5
    N, Cin, H, W = x_nchw.shape
    Cout = weight_oihw.shape[0]
    HW = H * W

    x = jnp.transpose(x_nchw, (0, 2, 3, 1))
    w_cat = jnp.transpose(weight_oihw.astype(jnp.float32),
                          (2, 1, 3, 0)).reshape(3 * Cin, 3 * Cout)
    w_cat = w_cat.astype(jnp.bfloat16)
    masks = jnp.asarray(_edge_masks(H, W, Cout), dtype=jnp.float32)

    conv_flops = 2 * Cout * 9 * Cin * HW

    y_raw, stats = pl.pallas_call(
        functools.partial(_conv_stats_kernel, H=H, W=W),
        out_shape=(
            jax.ShapeDtypeStruct((N, HW, Cout), jnp.bfloat16),
            jax.ShapeDtypeStruct((N, 2, Cout), jnp.float32),
        ),
        grid=(N,),
        in_specs=[
            pl.BlockSpec((1, H, W, Cin), lambda n: (n, 0, 0, 0)),
            pl.BlockSpec((3 * Cin, 3 * Cout), lambda n: (0, 0)),
            pl.BlockSpec((2, HW, Cout), lambda n: (0, 0, 0)),
        ],
        scratch_shapes=[pltpu.VMEM((HW, 3 * Cin), jnp.bfloat16)],
        out_specs=(
            pl.BlockSpec((1, HW, Cout), lambda n: (n, 0, 0)),
            pl.BlockSpec((1, 2, Cout), lambda n: (n, 0, 0)),
        ),
        compiler_params=pltpu.CompilerParams(
            dimension_semantics=("parallel",)),
        cost_estimate=pl.CostEstimate(
            flops=N * conv_flops,
            transcendentals=0,
            bytes_accessed=4 * N * Cin * HW + 2 * (3 * Cin * 3 * Cout
                                + N * Cout * HW) + 4 * (2 * HW * Cout
                                + N * Cout * 2)),
    )(x, w_cat, masks)

    y = pl.pallas_call(
        functools.partial(_norm_relu_kernel,
                          inv_count=1.0 / float(N * HW), eps=eps),
        out_shape=jax.ShapeDtypeStruct((N, HW, Cout), jnp.float32),
        grid=(N,),
        in_specs=[
            pl.BlockSpec((N, 2, Cout), lambda n: (0, 0, 0)),
            pl.BlockSpec((1, Cout), lambda n: (0, 0)),
            pl.BlockSpec((1, HW, Cout), lambda n: (n, 0, 0)),
        ],
        out_specs=pl.BlockSpec((1, HW, Cout), lambda n: (n, 0, 0)),
        compiler_params=pltpu.CompilerParams(
            dimension_semantics=("parallel",)),
        cost_estimate=pl.CostEstimate(
            flops=2 * N * Cout * HW,
            transcendentals=Cout,
            bytes_accessed=2 * N * Cout * HW + 4 * N * Cout * HW
                           + 4 * (N * Cout * 2 + Cout)),
    )(stats, beta.astype(jnp.float32).reshape(1, Cout), y_raw)

    return jnp.transpose(y.reshape(N, H, W, Cout), (0, 3, 1, 2))


def kernel(x_nchw, weight_oihw, beta):
    return _conv_block(x_nchw, weight_oihw, beta)


# speedup vs baseline: 3.4358x; 1.0021x over previous
import functools

import numpy as np

_IMGS_PER_STEP = 2

import jax
import jax.numpy as jnp
from jax import lax
from jax.experimental import pallas as pl
from jax.experimental.pallas import tpu as pltpu


def _edge_masks(H, W, C):
    col = np.arange(H * W) % W
    m = np.ones((2, H * W, 1), np.float32)
    m[0, col == 0, 0] = 0.0
    m[1, col == W - 1, 0] = 0.0
    return np.broadcast_to(m, (2, H * W, C)).copy()


def _conv_stats_kernel(x_ref, w_ref, m_ref, o_ref, s_ref, x3_ref, *, H, W):
    HW = H * W
    B = x_ref.shape[0]
    Cin = x_ref.shape[3]
    Cout = w_ref.shape[1] // 3
    zrow = jnp.zeros((W, Cin), jnp.float32)
    for b in range(B):
        x2 = x_ref[b].reshape(HW, Cin)
        x3_ref[:, 0:Cin] = jnp.concatenate(
            [zrow, x2[0:HW - W]], axis=0).astype(jnp.bfloat16)
        x3_ref[:, Cin:2 * Cin] = x2.astype(jnp.bfloat16)
        x3_ref[:, 2 * Cin:] = jnp.concatenate(
            [x2[W:HW], zrow], axis=0).astype(jnp.bfloat16)

        z = jnp.dot(x3_ref[...], w_ref[...],
                    preferred_element_type=jnp.float32)

        acc = z[:, Cout:2 * Cout]
        acc = acc + pltpu.roll(z[:, 0:Cout], 1, axis=0) * m_ref[0]
        acc = acc + pltpu.roll(z[:, 2 * Cout:], HW - 1, axis=0) * m_ref[1]

        o_ref[b] = acc.astype(jnp.bfloat16)
        s_ref[b, 0:1, :] = jnp.sum(acc, axis=0, keepdims=True)
        s_ref[b, 1:2, :] = jnp.sum(acc * acc, axis=0, keepdims=True)


def _norm_relu_kernel(s_ref, beta_ref, y_ref, o_ref, *, inv_count, eps):
    tot = jnp.sum(s_ref[...], axis=0)
    mean = tot[0:1, :] * inv_count
    var = jnp.maximum(tot[1:2, :] * inv_count - mean * mean, 0.0)
    scale = lax.rsqrt(var + eps)
    shift = beta_ref[...] - mean * scale
    y = y_ref[0].astype(jnp.float32)
    o_ref[...] = jnp.maximum(y * scale + shift, 0.0)[None]


@jax.jit
def _conv_block(x_nchw, weight_oihw, beta):
    eps = 1e-5
    N, Cin, H, W = x_nchw.shape
    Cout = weight_oihw.shape[0]
    HW = H * W

    x = jnp.transpose(x_nchw, (0, 2, 3, 1))
    w_cat = jnp.transpose(weight_oihw.astype(jnp.float32),
                          (2, 1, 3, 0)).reshape(3 * Cin, 3 * Cout)
    w_cat = w_cat.astype(jnp.bfloat16)
    masks = jnp.asarray(_edge_masks(H, W, Cout), dtype=jnp.float32)

    conv_flops = 2 * Cout * 9 * Cin * HW
    bs = _IMGS_PER_STEP if N % _IMGS_PER_STEP == 0 else 1

    y_raw, stats = pl.pallas_call(
        functools.partial(_conv_stats_kernel, H=H, W=W),
        out_shape=(
            jax.ShapeDtypeStruct((N, HW, Cout), jnp.bfloat16),
            jax.ShapeDtypeStruct((N, 2, Cout), jnp.float32),
        ),
        grid=(N // bs,),
        in_specs=[
            pl.BlockSpec((bs, H, W, Cin), lambda n: (n, 0, 0, 0)),
            pl.BlockSpec((3 * Cin, 3 * Cout), lambda n: (0, 0)),
            pl.BlockSpec((2, HW, Cout), lambda n: (0, 0, 0)),
        ],
        scratch_shapes=[pltpu.VMEM((HW, 3 * Cin), jnp.bfloat16)],
        out_specs=(
            pl.BlockSpec((bs, HW, Cout), lambda n: (n, 0, 0)),
            pl.BlockSpec((bs, 2, Cout), lambda n: (n, 0, 0)),
        ),
        compiler_params=pltpu.CompilerParams(
            dimension_semantics=("parallel",)),
        cost_estimate=pl.CostEstimate(
            flops=N * conv_flops,
            transcendentals=0,
            bytes_accessed=4 * N * Cin * HW + 2 * (3 * Cin * 3 * Cout
                                + N * Cout * HW) + 4 * (2 * HW * Cout
                                + N * Cout * 2)),
    )(x, w_cat, masks)

    y = pl.pallas_call(
        functools.partial(_norm_relu_kernel,
                          inv_count=1.0 / float(N * HW), eps=eps),
        out_shape=jax.ShapeDtypeStruct((N, HW, Cout), jnp.float32),
        grid=(N,),
        in_specs=[
            pl.BlockSpec((N, 2, Cout), lambda n: (0, 0, 0)),
            pl.BlockSpec((1, Cout), lambda n: (0, 0)),
            pl.BlockSpec((1, HW, Cout), lambda n: (n, 0, 0)),
        ],
        out_specs=pl.BlockSpec((1, HW, Cout), lambda n: (n, 0, 0)),
        compiler_params=pltpu.CompilerParams(
            dimension_semantics=("parallel",)),
        cost_estimate=pl.CostEstimate(
            flops=2 * N * Cout * HW,
            transcendentals=Cout,
            bytes_accessed=2 * N * Cout * HW + 4 * N * Cout * HW
                           + 4 * (N * Cout * 2 + Cout)),
    )(stats, beta.astype(jnp.float32).reshape(1, Cout), y_raw)

    return jnp.transpose(y.reshape(N, H, W, Cout), (0, 3, 1, 2))


def kernel(x_nchw, weight_oihw, beta):
    return _conv_block(x_nchw, weight_oihw, beta)
